# baseline (speedup 1.0000x reference)
import jax
import jax.numpy as jnp
from jax import lax
from jax.experimental import pallas as pl
from jax.experimental.pallas import tpu as pltpu
from jax.experimental.pallas import tpu_sc as plsc

VOCAB = 1000000
EMBED = 64
MAX_LEN = 200
BATCH = 4096
SEQ_LEN = 200

NUM_WORKERS = 32
TOTAL_ROWS = BATCH * SEQ_LEN
ROWS_PER_W = TOTAL_ROWS // NUM_WORKERS
CHUNK = 128
CHUNKS_PER_W = ROWS_PER_W // CHUNK
NBUF = 4
VREGS_PER_ROW = EMBED // 16

NFULL = VOCAB // 128
T1_STEPS = (NFULL + NUM_WORKERS - 1) // NUM_WORKERS


def _transpose_body(tT_hbm, tail_hbm, t2_hbm, s_in, s_out0, s_out1, w0, w1):
    wid = lax.axis_index("s") * 2 + lax.axis_index("c")
    s_out = (s_out0, s_out1)
    wsem = (w0, w1)

    iotav = lax.iota(jnp.int32, 16)
    rowj = [iotav + 16 * j for j in range(4)]

    def do_block(b, buf):
        pltpu.sync_copy(
            tT_hbm.at[:, pl.ds(pl.multiple_of(128 * b, 128), 128)], s_in)
        O = s_out[buf]

        def tr_row(r, _):
            col0 = jnp.full((16,), 2 * r, jnp.int32)
            col1 = col0 + 1
            for j in range(4):
                O[r, pl.ds(16 * j, 16)] = plsc.load_gather(
                    s_in, [rowj[j], col0])
            for j in range(4):
                O[r, pl.ds(64 + 16 * j, 16)] = plsc.load_gather(
                    s_in, [rowj[j], col1])
            return 0

        lax.fori_loop(0, 64, tr_row, 0, unroll=2)
        pltpu.async_copy(
            O, t2_hbm.at[pl.ds(pl.multiple_of(64 * b, 8), 64)], wsem[buf])

    def write_wait(b, buf):
        pltpu.make_async_copy(
            s_out[buf], t2_hbm.at[pl.ds(pl.multiple_of(64 * b, 8), 64)],
            wsem[buf]).wait()

    def step(tt, _):
        for buf in range(2):
            t = 2 * tt + buf
            b = wid + NUM_WORKERS * t
            bprev = b - 2 * NUM_WORKERS

            @pl.when((t >= 2) & (bprev < NFULL))
            def _(bprev=bprev, buf=buf):
                write_wait(bprev, buf)

            @pl.when((t < T1_STEPS) & (b < NFULL))
            def _(b=b, buf=buf):
                do_block(b, buf)
        return 0

    lax.fori_loop(0, T1_STEPS // 2 + 2, step, 0)

    @pl.when(wid == 0)
    def _():
        pltpu.sync_copy(tail_hbm, s_out0.at[pl.ds(0, 32)])
        pltpu.sync_copy(s_out0.at[pl.ds(0, 32)],
                        t2_hbm.at[pl.ds(NFULL * 64, 32)])


def _gather_body(seq_hbm, table_hbm, pos2_hbm, out_hbm,
                 idx_all, pos_v, r0, r1, r2, r3,
                 g0, g1, g2, g3, w0, w1, w2, w3):
    rows = (r0, r1, r2, r3)
    gsem = (g0, g1, g2, g3)
    wsem = (w0, w1, w2, w3)
    wid = lax.axis_index("s") * 2 + lax.axis_index("c")
    base = wid * ROWS_PER_W

    pltpu.sync_copy(seq_hbm.at[pl.ds(wid * CHUNKS_PER_W, CHUNKS_PER_W)],
                    idx_all)
    pltpu.sync_copy(pos2_hbm, pos_v)

    def gather_start(c, b):
        pltpu.async_copy(table_hbm.at[idx_all.at[c]], rows[b], gsem[b])

    def gather_wait(c, b):
        pltpu.make_async_copy(table_hbm.at[idx_all.at[c]], rows[b],
                              gsem[b]).wait()

    def write_start(c, b):
        pltpu.async_copy(rows[b], out_hbm.at[pl.ds(base + c * CHUNK, CHUNK)],
                         wsem[b])

    def write_wait(c, b):
        pltpu.make_async_copy(rows[b],
                              out_hbm.at[pl.ds(base + c * CHUNK, CHUNK)],
                              wsem[b]).wait()

    gather_start(0, 0)
    gather_start(1, 1)

    def outer(cc, _):
        for b in range(NBUF):
            c = NBUF * cc + b
            gather_wait(c, b)
            phase = (c * CHUNK) % MAX_LEN

            def add_body(r, _):
                pr = phase + r
                for j in range(VREGS_PER_ROW):
                    s = pl.ds(j * 16, 16)
                    rows[b][r, s] = rows[b][r, s] + pos_v[pr, s]
                return 0

            lax.fori_loop(0, CHUNK, add_body, 0, unroll=4)
            write_start(c, b)

            b2 = (b + 2) % NBUF

            @pl.when(c >= 2)
            def _():
                write_wait(c - 2, b2)

            @pl.when(c + 2 < CHUNKS_PER_W)
            def _():
                gather_start(c + 2, b2)
        return 0

    lax.fori_loop(0, CHUNKS_PER_W // NBUF, outer, 0)

    write_wait(CHUNKS_PER_W - 2, (CHUNKS_PER_W - 2) % NBUF)
    write_wait(CHUNKS_PER_W - 1, (CHUNKS_PER_W - 1) % NBUF)


def kernel(seq, token_table, pos_table):
    mesh = plsc.VectorSubcoreMesh(core_axis_name="c", subcore_axis_name="s")

    tT = token_table.T
    tail = token_table[NFULL * 128:].reshape(64 // 2, 128)
    t2 = pl.kernel(
        _transpose_body,
        out_type=jax.ShapeDtypeStruct((VOCAB // 2, 128), jnp.float32),
        mesh=mesh,
        scratch_types=[
            pltpu.VMEM((EMBED, 128), jnp.float32),
            pltpu.VMEM((EMBED, 128), jnp.float32),
            pltpu.VMEM((EMBED, 128), jnp.float32),
            pltpu.SemaphoreType.DMA,
            pltpu.SemaphoreType.DMA,
        ],
        compiler_params=pltpu.CompilerParams(use_tc_tiling_on_sc=True,
                                             needs_layout_passes=False),
    )(tT, tail)

    seq2d = seq.reshape(TOTAL_ROWS // CHUNK, CHUNK)
    pos2 = jnp.concatenate([pos_table, pos_table], axis=0)
    table = t2.reshape(VOCAB, EMBED)

    out = pl.kernel(
        _gather_body,
        out_type=jax.ShapeDtypeStruct((TOTAL_ROWS, EMBED), jnp.float32),
        mesh=mesh,
        scratch_types=[
            pltpu.VMEM((CHUNKS_PER_W, CHUNK), jnp.int32),
            pltpu.VMEM((2 * MAX_LEN, EMBED), jnp.float32),
        ] + [pltpu.VMEM((CHUNK, EMBED), jnp.float32)] * NBUF
          + [pltpu.SemaphoreType.DMA] * (2 * NBUF),
        compiler_params=pltpu.CompilerParams(use_tc_tiling_on_sc=False),
    )(seq2d, table, pos2)
    return out.reshape(BATCH, SEQ_LEN, EMBED)


# speedup vs baseline: 1.7996x; 1.7996x over previous
import jax
import jax.numpy as jnp
from jax import lax
from jax.experimental import pallas as pl
from jax.experimental.pallas import tpu as pltpu
from jax.experimental.pallas import tpu_sc as plsc

VOCAB = 1000000
EMBED = 64
MAX_LEN = 200
BATCH = 4096
SEQ_LEN = 200

NUM_WORKERS = 32

SLAB = 256
NFULL = (VOCAB // SLAB)
T1_STEPS = (NFULL + NUM_WORKERS - 1) // NUM_WORKERS

QB = 128
NQ = BATCH // QB
UNITS = SEQ_LEN * NQ
UNITS_PER_W = UNITS // NUM_WORKERS


def _transpose_body(tT_hbm, tail_hbm, t2_hbm,
                    s_in0, s_in1, s_out0, s_out1, i0, i1, w0, w1):
    wid = lax.axis_index("s") * 2 + lax.axis_index("c")
    s_in = (s_in0, s_in1)
    s_out = (s_out0, s_out1)
    isem = (i0, i1)
    wsem = (w0, w1)

    iotav = lax.iota(jnp.int32, 16)
    rowj = [iotav + 16 * j for j in range(4)]

    def in_start(b, buf):
        pltpu.async_copy(
            tT_hbm.at[:, pl.ds(pl.multiple_of(SLAB * b, 128), SLAB)],
            s_in[buf], isem[buf])

    def in_wait(b, buf):
        pltpu.make_async_copy(
            tT_hbm.at[:, pl.ds(pl.multiple_of(SLAB * b, 128), SLAB)],
            s_in[buf], isem[buf]).wait()

    def write_start(b, buf):
        pltpu.async_copy(
            s_out[buf],
            t2_hbm.at[pl.ds(pl.multiple_of(SLAB // 2 * b, 8), SLAB // 2)],
            wsem[buf])

    def write_wait(b, buf):
        pltpu.make_async_copy(
            s_out[buf],
            t2_hbm.at[pl.ds(pl.multiple_of(SLAB // 2 * b, 8), SLAB // 2)],
            wsem[buf]).wait()

    def do_block(buf):
        S = s_in[buf]
        O = s_out[buf]

        @plsc.parallel_loop(0, SLAB // 2, unroll=4)
        def _(r):
            c0 = jnp.full((16,), 2 * r, jnp.int32)
            c1 = c0 + 1
            for j in range(4):
                O[r, pl.ds(16 * j, 16)] = plsc.load_gather(S, [rowj[j], c0])
            for j in range(4):
                O[r, pl.ds(64 + 16 * j, 16)] = plsc.load_gather(
                    S, [rowj[j], c1])

    in_start(wid, 0)
    in_start(wid + NUM_WORKERS, 1)

    def step(tt, _):
        for buf in range(2):
            t = 2 * tt + buf
            b = wid + NUM_WORKERS * t
            bprev = b - 2 * NUM_WORKERS

            @pl.when((t >= 2) & (bprev < NFULL))
            def _(bprev=bprev, buf=buf):
                write_wait(bprev, buf)

            @pl.when(b < NFULL)
            def _(b=b, buf=buf):
                in_wait(b, buf)
                do_block(buf)
                write_start(b, buf)
                bnext = b + 2 * NUM_WORKERS

                @pl.when(bnext < NFULL)
                def _():
                    in_start(bnext, buf)
        return 0

    lax.fori_loop(0, T1_STEPS // 2 + 2, step, 0)

    @pl.when(wid == 0)
    def _():
        pltpu.sync_copy(tail_hbm, s_out0.at[pl.ds(0, 32)])
        pltpu.sync_copy(s_out0.at[pl.ds(0, 32)],
                        t2_hbm.at[pl.ds(NFULL * (SLAB // 2), 32)])


def _gather_body(seqT_hbm, table_hbm, pos_hbm, out_hbm,
                 idx_all, pos_v, g0, g1, p0, p1,
                 gs0, gs1, ws0, ws1):
    gbuf = (g0, g1)
    pbuf = (p0, p1)
    gsem = (gs0, gs1)
    wsem = (ws0, ws1)
    wid = lax.axis_index("s") * 2 + lax.axis_index("c")
    ubase = wid * UNITS_PER_W

    pltpu.sync_copy(seqT_hbm.at[pl.ds(wid * UNITS_PER_W, UNITS_PER_W)],
                    idx_all)
    pltpu.sync_copy(pos_hbm, pos_v)

    iotav = lax.iota(jnp.int32, 16)
    rowg = [iotav + 16 * g for g in range(8)]

    def gather_start(uu, buf):
        pltpu.async_copy(table_hbm.at[idx_all.at[uu]], gbuf[buf], gsem[buf])

    def gather_wait(uu, buf):
        pltpu.make_async_copy(table_hbm.at[idx_all.at[uu]], gbuf[buf],
                              gsem[buf]).wait()

    def write_start(uu, buf):
        U = ubase + uu
        l = U // NQ
        q = U % NQ
        pltpu.async_copy(pbuf[buf], out_hbm.at[l, :, q], wsem[buf])

    def write_wait(uu, buf):
        U = ubase + uu
        l = U // NQ
        q = U % NQ
        pltpu.make_async_copy(pbuf[buf], out_hbm.at[l, :, q],
                              wsem[buf]).wait()

    def transform(uu, buf):
        U = ubase + uu
        l = U // NQ
        G = gbuf[buf]
        P = pbuf[buf]
        l16 = jnp.full((16,), l, jnp.int32)

        @plsc.parallel_loop(0, EMBED, unroll=2)
        def _(e):
            e16 = jnp.full((16,), e, jnp.int32)
            pv = plsc.load_gather(pos_v, [l16, e16])
            eh = e // 8
            el = e % 8
            for g in range(8):
                v = plsc.load_gather(G, [rowg[g], e16]) + pv
                P[eh, el, pl.ds(16 * g, 16)] = v

    gather_start(0, 0)
    gather_start(1, 1)

    def outer(cc, _):
        for buf in range(2):
            uu = 2 * cc + buf
            gather_wait(uu, buf)

            @pl.when(uu >= 2)
            def _(uu=uu, buf=buf):
                write_wait(uu - 2, buf)

            transform(uu, buf)
            write_start(uu, buf)

            @pl.when(uu + 2 < UNITS_PER_W)
            def _(uu=uu, buf=buf):
                gather_start(uu + 2, buf)
        return 0

    lax.fori_loop(0, UNITS_PER_W // 2, outer, 0)

    write_wait(UNITS_PER_W - 2, 0)
    write_wait(UNITS_PER_W - 1, 1)


def kernel(seq, token_table, pos_table):
    mesh = plsc.VectorSubcoreMesh(core_axis_name="c", subcore_axis_name="s")

    tT = token_table.T
    tail = token_table[NFULL * SLAB:].reshape(64 // 2, 128)
    t2 = pl.kernel(
        _transpose_body,
        out_type=jax.ShapeDtypeStruct((VOCAB // 2, 128), jnp.float32),
        mesh=mesh,
        scratch_types=[
            pltpu.VMEM((EMBED, SLAB), jnp.float32),
            pltpu.VMEM((EMBED, SLAB), jnp.float32),
            pltpu.VMEM((SLAB // 2, 128), jnp.float32),
            pltpu.VMEM((SLAB // 2, 128), jnp.float32),
            pltpu.SemaphoreType.DMA,
            pltpu.SemaphoreType.DMA,
            pltpu.SemaphoreType.DMA,
            pltpu.SemaphoreType.DMA,
        ],
        compiler_params=pltpu.CompilerParams(use_tc_tiling_on_sc=True,
                                             needs_layout_passes=False),
    )(tT, tail)

    seqT = seq.T.reshape(UNITS, QB)
    table = t2.reshape(VOCAB, EMBED)
    k5 = pl.kernel(
        _gather_body,
        out_type=jax.ShapeDtypeStruct((SEQ_LEN, 8, NQ, 8, QB), jnp.float32),
        mesh=mesh,
        scratch_types=[
            pltpu.VMEM((UNITS_PER_W, QB), jnp.int32),
            pltpu.VMEM((MAX_LEN, EMBED), jnp.float32),
            pltpu.VMEM((QB, EMBED), jnp.float32),
            pltpu.VMEM((QB, EMBED), jnp.float32),
            pltpu.VMEM((8, 8, QB), jnp.float32),
            pltpu.VMEM((8, 8, QB), jnp.float32),
            pltpu.SemaphoreType.DMA,
            pltpu.SemaphoreType.DMA,
            pltpu.SemaphoreType.DMA,
            pltpu.SemaphoreType.DMA,
        ],
        compiler_params=pltpu.CompilerParams(use_tc_tiling_on_sc=False,
                                             needs_layout_passes=False),
    )(seqT, table, pos_table)
    return k5.transpose(2, 4, 0, 1, 3).reshape(BATCH, SEQ_LEN, EMBED)
